# Initial kernel scaffold; baseline (speedup 1.0000x reference)
import jax
import jax.numpy as jnp
from jax.experimental import pallas as pl


def kernel(fn, track_pad_mask, conv_w, conv_b, fc_w, fc_b, mu_w, mu_b, codebook):
    raise NotImplementedError("write your pallas kernel here")


# speedup vs baseline: 2.3392x; 2.3392x over previous
import functools

import jax
import jax.numpy as jnp
from jax.experimental import pallas as pl
from jax.experimental.pallas import tpu as pltpu

BS = 16384
L = 32
P = 8
KSZ = 4
NUM_CH = 64
EMB_SIZE = 512
Z_DIM = 256
NUM_CODES = 128
COMMIT = 0.25

BLK = 1024
N_BLK = BS // BLK


def _fused_body(fn_ref, valid_ref, wc_ref, cb_tiled_ref, codebook_ref, cbt_ref,
                fc_wpt_ref, fc_b_ref, mu_wt_ref, mu_b_ref,
                mu_ref, cmt_ref, perp_ref,
                hist_ref, acc_ref):
    i = pl.program_id(0)

    @pl.when(i == 0)
    def _init():
        hist_ref[...] = jnp.zeros_like(hist_ref)
        acc_ref[0] = 0.0
        acc_ref[1] = 0.0

    fn = fn_ref[...]
    valid = valid_ref[...]
    zbig = jnp.maximum(
        jnp.dot(fn, wc_ref[...], preferred_element_type=jnp.float32)
        + cb_tiled_ref[...], 0.0)

    codebook = codebook_ref[...]
    cbn2 = jnp.sum(codebook * codebook, axis=1)[None, :]
    iota = jax.lax.broadcasted_iota(jnp.int32, (BLK, NUM_CODES), 1)

    h = jnp.zeros((BLK, EMB_SIZE), jnp.float32)
    hist = jnp.zeros((1, NUM_CODES), jnp.float32)
    dsum = 0.0
    for p in range(P):
        z_p = zbig[:, p * NUM_CH:(p + 1) * NUM_CH]
        zn2 = jnp.sum(z_p * z_p, axis=1, keepdims=True)
        s_p = jnp.dot(z_p, cbt_ref[...], preferred_element_type=jnp.float32)
        dist = zn2 + cbn2 - 2.0 * s_p
        dmin = jnp.min(dist, axis=1, keepdims=True)
        idx = jnp.min(jnp.where(dist == dmin, iota, NUM_CODES), axis=1,
                      keepdims=True)
        oh = (iota == idx).astype(jnp.float32)
        hist = hist + jnp.sum(oh, axis=0, keepdims=True)
        dsum = dsum + jnp.sum(dmin * valid)
        q_p = jnp.dot(oh, codebook, preferred_element_type=jnp.float32)
        h = h + jnp.dot(q_p, fc_wpt_ref[p], preferred_element_type=jnp.float32)

    h = h + fc_b_ref[...]
    mu_ref[...] = (jnp.dot(h, mu_wt_ref[...], preferred_element_type=jnp.float32)
                   + mu_b_ref[...])

    hist_ref[...] += hist
    acc_ref[0] += dsum
    acc_ref[1] += jnp.sum(valid)

    @pl.when(i == N_BLK - 1)
    def _fini():
        denom = jnp.maximum(acc_ref[1] * (P * NUM_CH), 1.0)
        cmt_ref[...] = jnp.full((1, 1), COMMIT * acc_ref[0] / denom,
                                jnp.float32)
        avgp = hist_ref[...] / float(BS * P)
        perp_ref[...] = jnp.full(
            (1, 1), jnp.exp(-jnp.sum(avgp * jnp.log(avgp + 1e-10))),
            jnp.float32)


@jax.jit
def kernel(fn, track_pad_mask, conv_w, conv_b, fc_w, fc_b, mu_w, mu_b, codebook):
    valid = 1.0 - track_pad_mask.astype(jnp.float32)
    w_kc = conv_w[:, 0, :].T
    wc = jnp.kron(jnp.eye(P, dtype=jnp.float32), w_kc)
    cb_tiled = jnp.tile(conv_b, P)[None, :]
    fc_wpt = fc_w.reshape(EMB_SIZE, NUM_CH, P).transpose(2, 1, 0)

    mu, cmt, perp = pl.pallas_call(
        _fused_body,
        grid=(N_BLK,),
        in_specs=[
            pl.BlockSpec((BLK, L), lambda i: (i, 0)),
            pl.BlockSpec((BLK, 1), lambda i: (i, 0)),
            pl.BlockSpec((L, EMB_SIZE), lambda i: (0, 0)),
            pl.BlockSpec((1, EMB_SIZE), lambda i: (0, 0)),
            pl.BlockSpec((NUM_CODES, NUM_CH), lambda i: (0, 0)),
            pl.BlockSpec((NUM_CH, NUM_CODES), lambda i: (0, 0)),
            pl.BlockSpec((P, NUM_CH, EMB_SIZE), lambda i: (0, 0, 0)),
            pl.BlockSpec((1, EMB_SIZE), lambda i: (0, 0)),
            pl.BlockSpec((EMB_SIZE, Z_DIM), lambda i: (0, 0)),
            pl.BlockSpec((1, Z_DIM), lambda i: (0, 0)),
        ],
        out_specs=[
            pl.BlockSpec((BLK, Z_DIM), lambda i: (i, 0)),
            pl.BlockSpec((1, 1), lambda i: (0, 0)),
            pl.BlockSpec((1, 1), lambda i: (0, 0)),
        ],
        out_shape=[
            jax.ShapeDtypeStruct((BS, Z_DIM), jnp.float32),
            jax.ShapeDtypeStruct((1, 1), jnp.float32),
            jax.ShapeDtypeStruct((1, 1), jnp.float32),
        ],
        scratch_shapes=[
            pltpu.VMEM((1, NUM_CODES), jnp.float32),
            pltpu.SMEM((2,), jnp.float32),
        ],
        compiler_params=pltpu.CompilerParams(
            dimension_semantics=("arbitrary",)),
    )(fn, valid, wc, cb_tiled, codebook, codebook.T, fc_wpt,
      fc_b[None, :], mu_w.T, mu_b[None, :])

    return mu, cmt.reshape(()), perp.reshape(())
